# baseline (speedup 1.0000x reference)
import jax
import jax.numpy as jnp
from jax import lax
from jax.experimental import pallas as pl
from jax.experimental.pallas import tpu as pltpu
from jax.experimental.pallas import tpu_sc as plsc

NUM_LANES = 16
NUM_WORKERS = 16
SC_COLS = 8192
SC_CHUNK = 2048
TC_BLOCK = 8192


def _take16(vec, idx):
    dnums = lax.GatherDimensionNumbers(
        offset_dims=(), collapsed_slice_dims=(0,), start_index_map=(0,))
    return lax.gather(vec, idx[:, None], dnums, slice_sizes=(1,),
                      mode=lax.GatherScatterMode.PROMISE_IN_BOUNDS)


def _sc_body(rows_per_w, n_chunks):
    def body(x_hbm, knots_hbm, temp_hbm, out_hbm,
             tab_v, in0, in1, out0, out1,
             si0, si1, so0, so1):
        nc = 1
        wid = lax.axis_index("s") * nc + lax.axis_index("c")
        row0 = wid * rows_per_w

        pltpu.sync_copy(knots_hbm, tab_v.at[pl.ds(0, 8)])
        pltpu.sync_copy(temp_hbm, tab_v.at[pl.ds(8, 1)])
        tab = tab_v[...]
        iota = lax.iota(jnp.int32, NUM_LANES)
        temp = _take16(tab, jnp.full((NUM_LANES,), 8, jnp.int32))
        neg_invt = -1.0 / (temp + 1e-6)
        knot_hi = _take16(tab, jnp.minimum(iota + 1, 7))
        bvec = knot_hi - tab
        avec = tab - iota.astype(jnp.float32) * bvec

        def compute(inb, outb):
            @plsc.parallel_loop(0, SC_CHUNK, step=NUM_LANES, unroll=2)
            def _(i):
                for r in range(rows_per_w):
                    x = inb[r, pl.ds(i, NUM_LANES)]
                    e = jnp.exp(x * neg_invt)
                    idxf = 7.0 / (1.0 + e)
                    ii = idxf.astype(jnp.int32)
                    outb[r, pl.ds(i, NUM_LANES)] = (
                        _take16(avec, ii) + _take16(bvec, ii) * idxf)

        in_bufs, out_bufs = (in0, in1), (out0, out1)
        in_sems, out_sems = (si0, si1), (so0, so1)
        cin, cout = {}, {}
        cin[0] = pltpu.async_copy(
            x_hbm.at[pl.ds(row0, rows_per_w), pl.ds(0, SC_CHUNK)],
            in_bufs[0], in_sems[0])
        for ci in range(n_chunks):
            if ci + 1 < n_chunks:
                cin[ci + 1] = pltpu.async_copy(
                    x_hbm.at[pl.ds(row0, rows_per_w),
                             pl.ds((ci + 1) * SC_CHUNK, SC_CHUNK)],
                    in_bufs[(ci + 1) % 2], in_sems[(ci + 1) % 2])
            cin[ci].wait()
            if ci >= 2:
                cout[ci - 2].wait()
            compute(in_bufs[ci % 2], out_bufs[ci % 2])
            cout[ci] = pltpu.async_copy(
                out_bufs[ci % 2],
                out_hbm.at[pl.ds(row0, rows_per_w),
                           pl.ds(ci * SC_CHUNK, SC_CHUNK)],
                out_sems[ci % 2])
        if n_chunks >= 2:
            cout[n_chunks - 2].wait()
        cout[n_chunks - 1].wait()
    return body


def _tc_body(tab_ref, x_ref, out_ref):
    f = 7.0 / (1.0 + jnp.exp2(x_ref[...] * tab_ref[0, 8]))
    acc = tab_ref[0, 0] + tab_ref[0, 9] * f
    for i in range(1, 7):
        acc = acc + tab_ref[0, 9 + i] * jnp.maximum(f - float(i), 0.0)
    out_ref[...] = acc


def kernel(scores, knot_values, temperature):
    n_rows, n_cols = scores.shape
    xt = scores.T
    rows_per_w = n_cols // NUM_WORKERS
    n_sc_chunks = SC_COLS // SC_CHUNK
    n_tc_blocks = (n_rows - SC_COLS) // TC_BLOCK
    assert n_cols % NUM_WORKERS == 0 and SC_COLS % SC_CHUNK == 0
    assert (n_rows - SC_COLS) % TC_BLOCK == 0 and SC_COLS % TC_BLOCK == 0

    mesh = plsc.VectorSubcoreMesh(
        core_axis_name="c", subcore_axis_name="s", num_cores=1)
    sc_run = pl.kernel(
        _sc_body(rows_per_w, n_sc_chunks),
        out_type=jax.ShapeDtypeStruct((n_cols, SC_COLS), jnp.float32),
        mesh=mesh,
        scratch_types=[
            pltpu.VMEM((NUM_LANES,), jnp.float32),
            pltpu.VMEM((rows_per_w, SC_CHUNK), jnp.float32),
            pltpu.VMEM((rows_per_w, SC_CHUNK), jnp.float32),
            pltpu.VMEM((rows_per_w, SC_CHUNK), jnp.float32),
            pltpu.VMEM((rows_per_w, SC_CHUNK), jnp.float32),
            pltpu.SemaphoreType.DMA,
            pltpu.SemaphoreType.DMA,
            pltpu.SemaphoreType.DMA,
            pltpu.SemaphoreType.DMA,
        ],
    )
    sc_out = sc_run(xt, knot_values, temperature)

    neg_log2e_invt = -1.4426950408889634 / (temperature + 1e-6)
    b = knot_values[1:] - knot_values[:-1]
    d = jnp.concatenate([b[:1], b[1:] - b[:-1]])
    tab_tc = jnp.concatenate([knot_values, neg_log2e_invt, d]).reshape(1, 16)

    tc_run = pl.pallas_call(
        _tc_body,
        grid=(n_tc_blocks,),
        in_specs=[
            pl.BlockSpec((1, 16), lambda i: (0, 0), memory_space=pltpu.SMEM),
            pl.BlockSpec((n_cols, TC_BLOCK),
                         lambda i: (0, i + SC_COLS // TC_BLOCK)),
        ],
        out_specs=pl.BlockSpec((n_cols, TC_BLOCK),
                               lambda i: (0, i + SC_COLS // TC_BLOCK)),
        out_shape=jax.ShapeDtypeStruct((n_cols, n_rows), jnp.float32),
    )
    tc_out = tc_run(tab_tc, xt)

    merged = lax.dynamic_update_slice(tc_out, sc_out, (0, 0))
    return merged.T


# speedup vs baseline: 1.0807x; 1.0807x over previous
import jax
import jax.numpy as jnp
from jax import lax
from jax.experimental import pallas as pl
from jax.experimental.pallas import tpu as pltpu
from jax.experimental.pallas import tpu_sc as plsc

NUM_LANES = 16
NUM_WORKERS = 32
SC_COLS = 8192
SC_CHUNK = 2048
TC_BLOCK = 8192


def _take16(vec, idx):
    dnums = lax.GatherDimensionNumbers(
        offset_dims=(), collapsed_slice_dims=(0,), start_index_map=(0,))
    return lax.gather(vec, idx[:, None], dnums, slice_sizes=(1,),
                      mode=lax.GatherScatterMode.PROMISE_IN_BOUNDS)


def _sc_body(rows_per_w, n_chunks):
    def body(x_hbm, knots_hbm, temp_hbm, out_hbm,
             tab_v, in0, in1, out0, out1,
             si0, si1, so0, so1):
        nc = 2
        wid = lax.axis_index("s") * nc + lax.axis_index("c")
        row0 = wid * rows_per_w

        pltpu.sync_copy(knots_hbm, tab_v.at[pl.ds(0, 8)])
        pltpu.sync_copy(temp_hbm, tab_v.at[pl.ds(8, 1)])
        tab = tab_v[...]
        iota = lax.iota(jnp.int32, NUM_LANES)
        temp = _take16(tab, jnp.full((NUM_LANES,), 8, jnp.int32))
        neg_invt = -1.0 / (temp + 1e-6)
        knot_hi = _take16(tab, jnp.minimum(iota + 1, 7))
        bvec = knot_hi - tab
        avec = tab - iota.astype(jnp.float32) * bvec

        def compute(inb, outb):
            @plsc.parallel_loop(0, SC_CHUNK, step=NUM_LANES, unroll=2)
            def _(i):
                for r in range(rows_per_w):
                    x = inb[r, pl.ds(i, NUM_LANES)]
                    e = jnp.exp(x * neg_invt)
                    idxf = 7.0 / (1.0 + e)
                    ii = idxf.astype(jnp.int32)
                    outb[r, pl.ds(i, NUM_LANES)] = (
                        _take16(avec, ii) + _take16(bvec, ii) * idxf)

        in_bufs, out_bufs = (in0, in1), (out0, out1)
        in_sems, out_sems = (si0, si1), (so0, so1)
        cin, cout = {}, {}
        cin[0] = pltpu.async_copy(
            x_hbm.at[pl.ds(row0, rows_per_w), pl.ds(0, SC_CHUNK)],
            in_bufs[0], in_sems[0])
        for ci in range(n_chunks):
            if ci + 1 < n_chunks:
                cin[ci + 1] = pltpu.async_copy(
                    x_hbm.at[pl.ds(row0, rows_per_w),
                             pl.ds((ci + 1) * SC_CHUNK, SC_CHUNK)],
                    in_bufs[(ci + 1) % 2], in_sems[(ci + 1) % 2])
            cin[ci].wait()
            if ci >= 2:
                cout[ci - 2].wait()
            compute(in_bufs[ci % 2], out_bufs[ci % 2])
            cout[ci] = pltpu.async_copy(
                out_bufs[ci % 2],
                out_hbm.at[pl.ds(row0, rows_per_w),
                           pl.ds(ci * SC_CHUNK, SC_CHUNK)],
                out_sems[ci % 2])
        if n_chunks >= 2:
            cout[n_chunks - 2].wait()
        cout[n_chunks - 1].wait()
    return body


def _tc_body(tab_ref, x_ref, out_ref):
    f = 7.0 / (1.0 + jnp.exp2(x_ref[...] * tab_ref[0, 8]))
    acc = tab_ref[0, 0] + tab_ref[0, 9] * f
    for i in range(1, 7):
        acc = acc + tab_ref[0, 9 + i] * jnp.maximum(f - float(i), 0.0)
    out_ref[...] = acc


def kernel(scores, knot_values, temperature):
    n_rows, n_cols = scores.shape
    xt = scores.T
    rows_per_w = n_cols // NUM_WORKERS
    n_sc_chunks = SC_COLS // SC_CHUNK
    n_tc_blocks = (n_rows - SC_COLS) // TC_BLOCK
    assert n_cols % NUM_WORKERS == 0 and SC_COLS % SC_CHUNK == 0
    assert (n_rows - SC_COLS) % TC_BLOCK == 0 and SC_COLS % TC_BLOCK == 0

    mesh = plsc.VectorSubcoreMesh(core_axis_name="c", subcore_axis_name="s")
    sc_run = pl.kernel(
        _sc_body(rows_per_w, n_sc_chunks),
        out_type=jax.ShapeDtypeStruct((n_cols, SC_COLS), jnp.float32),
        mesh=mesh,
        scratch_types=[
            pltpu.VMEM((NUM_LANES,), jnp.float32),
            pltpu.VMEM((rows_per_w, SC_CHUNK), jnp.float32),
            pltpu.VMEM((rows_per_w, SC_CHUNK), jnp.float32),
            pltpu.VMEM((rows_per_w, SC_CHUNK), jnp.float32),
            pltpu.VMEM((rows_per_w, SC_CHUNK), jnp.float32),
            pltpu.SemaphoreType.DMA,
            pltpu.SemaphoreType.DMA,
            pltpu.SemaphoreType.DMA,
            pltpu.SemaphoreType.DMA,
        ],
    )
    sc_out = sc_run(xt, knot_values, temperature)

    neg_log2e_invt = -1.4426950408889634 / (temperature + 1e-6)
    b = knot_values[1:] - knot_values[:-1]
    d = jnp.concatenate([b[:1], b[1:] - b[:-1]])
    tab_tc = jnp.concatenate([knot_values, neg_log2e_invt, d]).reshape(1, 16)

    tc_run = pl.pallas_call(
        _tc_body,
        grid=(n_tc_blocks,),
        in_specs=[
            pl.BlockSpec((1, 16), lambda i: (0, 0), memory_space=pltpu.SMEM),
            pl.BlockSpec((n_cols, TC_BLOCK),
                         lambda i: (0, i + SC_COLS // TC_BLOCK)),
        ],
        out_specs=pl.BlockSpec((n_cols, TC_BLOCK),
                               lambda i: (0, i + SC_COLS // TC_BLOCK)),
        out_shape=jax.ShapeDtypeStruct((n_cols, n_rows), jnp.float32),
    )
    tc_out = tc_run(tab_tc, xt)

    merged = lax.dynamic_update_slice(tc_out, sc_out, (0, 0))
    return merged.T
